# baseline (speedup 1.0000x reference)
import functools

import jax
import jax.numpy as jnp
from jax import lax
from jax.experimental import pallas as pl
from jax.experimental.pallas import tpu as pltpu
from jax.experimental.pallas import tpu_sc as plsc

H = 16
TAB = 257
SEQ = 2048
PAD = 1919
SU = 4088
NQ = 31
NCLS = 16
NK = 16


def _build_s8(bias):
    def body(bias_ref, s8_ref):
        b = bias_ref[...]
        left = jnp.broadcast_to(b[:, :1], (H, PAD))
        right = jnp.broadcast_to(b[:, TAB - 1 : TAB], (H, PAD))
        g = jnp.concatenate([left, b, right], axis=1)
        for sh in range(8):
            s8_ref[:, sh, :] = g[:, 7 - sh : 7 - sh + SU]

    return pl.pallas_call(
        body,
        out_shape=jax.ShapeDtypeStruct((H, 8, SU), jnp.float32),
    )(bias)


def _materialize(s8):
    mesh = plsc.VectorSubcoreMesh(core_axis_name="c", subcore_axis_name="s")

    @functools.partial(
        pl.kernel,
        mesh=mesh,
        out_type=jax.ShapeDtypeStruct((H, SEQ // 8, SEQ // 128, 8, 128), jnp.float32),
        scratch_types=[
            pltpu.VMEM((2, 2, NQ, 8, 128), jnp.float32),
            pltpu.SemaphoreType.DMA,
            pltpu.SemaphoreType.DMA,
            pltpu.SemaphoreType.DMA,
        ],
        compiler_params=pltpu.CompilerParams(use_tc_tiling_on_sc=False),
    )
    def body(s8_hbm, out_hbm, l_vmem, sem_stage, sem_out, sem_out2):
        h = lax.axis_index("s")
        c = lax.axis_index("c")

        def stage_one(pr, pi, q):
            m0 = c * 8 + 2 * pr
            r0 = pl.multiple_of(120 - 8 * m0 - 8 * pi, 8)
            return pltpu.make_async_copy(
                s8_hbm.at[h, :, pl.ds(128 * q + r0, 128)],
                l_vmem.at[pr % 2, pi, q],
                sem_stage,
            )

        def stage_start(pr):
            def go(t, carry):
                stage_one(pr, t // NQ, t % NQ).start()
                return carry

            lax.fori_loop(0, 2 * NQ, go, 0)

        def stage_wait(pr):
            def go(t, carry):
                stage_one(pr, t // NQ, t % NQ).wait()
                return carry

            lax.fori_loop(0, 2 * NQ, go, 0)

        def out_one(pr, k, sem):
            m0 = c * 8 + 2 * pr
            return pltpu.make_async_copy(
                l_vmem.at[pr % 2, :, pl.ds(15 - k, NK)],
                out_hbm.at[h, pl.ds(m0 + NCLS * k, 2)],
                sem,
            )

        def out_start(pr):
            def go(k, carry):
                out_one(pr, 2 * k, sem_out).start()
                out_one(pr, 2 * k + 1, sem_out2).start()
                return carry

            lax.fori_loop(0, NK // 2, go, 0)

        def out_wait(pr):
            def go(k, carry):
                out_one(pr, 2 * k, sem_out).wait()
                out_one(pr, 2 * k + 1, sem_out2).wait()
                return carry

            lax.fori_loop(0, NK // 2, go, 0)

        stage_start(0)
        for pr in range(4):
            stage_wait(pr)
            out_start(pr)
            if pr < 3:
                stage_start(pr + 1)
            out_wait(pr)

    return body(s8)


def kernel(bias, qlen, klen):
    del qlen, klen
    out5 = _materialize(_build_s8(bias))
    return out5.transpose(0, 1, 3, 2, 4).reshape(H, SEQ, SEQ)


# speedup vs baseline: 1.0318x; 1.0318x over previous
import functools

import jax
import jax.numpy as jnp
from jax import lax
from jax.experimental import pallas as pl
from jax.experimental.pallas import tpu as pltpu
from jax.experimental.pallas import tpu_sc as plsc

H = 16
TAB = 257
SEQ = 2048
PAD = 1919
PADR = 1927
SU = 4096
NQ = 31
NCLS = 16
NK = 16


def _build_s8(bias):
    def body(bias_ref, s8_ref):
        b = bias_ref[...]
        left = jnp.broadcast_to(b[:, :1], (H, PAD))
        right = jnp.broadcast_to(b[:, TAB - 1 : TAB], (H, PADR))
        g = jnp.concatenate([left, b, right], axis=1)
        for sh in range(8):
            for uc in range(32):
                s8_ref[:, sh, uc, :] = g[:, 7 - sh + 128 * uc : 7 - sh + 128 * uc + 128]

    out4 = pl.pallas_call(
        body,
        out_shape=jax.ShapeDtypeStruct((H, 8, 32, 128), jnp.float32),
    )(bias)
    return out4.reshape(H, 8, SU)


def _materialize(s8):
    mesh = plsc.VectorSubcoreMesh(core_axis_name="c", subcore_axis_name="s")

    @functools.partial(
        pl.kernel,
        mesh=mesh,
        out_type=jax.ShapeDtypeStruct((H, SEQ // 8, SEQ // 128, 8, 128), jnp.float32),
        scratch_types=[
            pltpu.VMEM((2, 2, NQ, 8, 128), jnp.float32),
            pltpu.SemaphoreType.DMA,
            pltpu.SemaphoreType.DMA,
        ],
        compiler_params=pltpu.CompilerParams(use_tc_tiling_on_sc=False),
    )
    def body(s8_hbm, out_hbm, l_vmem, sem_stage, sem_out):
        h = lax.axis_index("s")
        c = lax.axis_index("c")

        def stage_one(pr, pi, q):
            m0 = c * 8 + 2 * pr
            r0 = pl.multiple_of(120 - 8 * m0 - 8 * pi, 8)
            return pltpu.make_async_copy(
                s8_hbm.at[h, :, pl.ds(128 * q + r0, 128)],
                l_vmem.at[pr % 2, pi, q],
                sem_stage,
            )

        def stage_start(pr):
            def go(t, carry):
                stage_one(pr, t // NQ, t % NQ).start()
                return carry

            lax.fori_loop(0, 2 * NQ, go, 0)

        def stage_wait(pr):
            def go(t, carry):
                stage_one(pr, t // NQ, t % NQ).wait()
                return carry

            lax.fori_loop(0, 2 * NQ, go, 0)

        def out_one(pr, k):
            m0 = c * 8 + 2 * pr
            return pltpu.make_async_copy(
                l_vmem.at[pr % 2, :, pl.ds(15 - k, NK)],
                out_hbm.at[h, pl.ds(m0 + NCLS * k, 2)],
                sem_out,
            )

        def out_start(pr):
            def go(k, carry):
                out_one(pr, k).start()
                return carry

            lax.fori_loop(0, NK, go, 0)

        def out_wait(pr):
            def go(k, carry):
                out_one(pr, k).wait()
                return carry

            lax.fori_loop(0, NK, go, 0)

        stage_start(0)
        for pr in range(4):
            stage_wait(pr)
            out_start(pr)
            if pr < 3:
                stage_start(pr + 1)
            out_wait(pr)

    return body(s8)


def kernel(bias, qlen, klen):
    del qlen, klen
    out5 = _materialize(_build_s8(bias))
    return out5.transpose(0, 1, 3, 2, 4).reshape(H, SEQ, SEQ)
